# baseline (speedup 1.0000x reference)
import jax
import jax.numpy as jnp
from jax.experimental import pallas as pl

B, N = 32, 32
NODE_F, EDGE_F, MSG, PASSES, OUT_F = 128, 16, 128, 3, 128
BN = B * N
E_ROWS = B * N * N


def _mpnn_kernel(a_ref, h_ref, e_ref, wmsg_ref, wupd_ref, wout_ref, out_ref):
    Af = a_ref[:]
    H0 = h_ref[:]
    W1 = wmsg_ref[0:NODE_F, :]
    W2 = wmsg_ref[NODE_F:2 * NODE_F, :]
    W3 = wmsg_ref[2 * NODE_F:, :]
    Wu1 = wupd_ref[0:NODE_F, :]
    Wu2 = wupd_ref[NODE_F:, :]
    Wo1 = wout_ref[0:NODE_F, :]
    Wo2 = wout_ref[NODE_F:, :]

    f32 = jnp.float32
    t_row = jax.lax.broadcasted_iota(jnp.int32, (N, BN), 0)
    t_col = jax.lax.broadcasted_iota(jnp.int32, (N, BN), 1)
    T = (t_col % N == t_row).astype(f32)
    D = (t_col // N == t_row).astype(f32)

    ea_parts = []
    for b in range(B):
        A_b = Af[b * N:(b + 1) * N, :]
        M_b = jnp.dot(A_b, T, preferred_element_type=f32) * D
        E_b = e_ref[b * N * N:(b + 1) * N * N, :]
        ea_parts.append(jnp.dot(M_b, E_b, preferred_element_type=f32))
    EA = jnp.concatenate(ea_parts, axis=0)
    E3 = jnp.dot(EA, W3, preferred_element_type=f32)

    deg = jnp.sum(Af, axis=1, keepdims=True)
    maskb = deg > 0.0
    maskf = maskb.astype(f32)
    A3 = Af.reshape(B, N, N)

    H = H0
    for _ in range(PASSES):
        Hb = H.reshape(B, N, NODE_F)
        neigh = jax.lax.dot_general(
            A3, Hb, (((2,), (1,)), ((0,), (0,))),
            preferred_element_type=f32).reshape(BN, NODE_F)
        msg = deg * jnp.dot(H, W1, preferred_element_type=f32) \
            + jnp.dot(neigh, W2, preferred_element_type=f32) + E3
        new = jnp.tanh(jnp.dot(H, Wu1, preferred_element_type=f32)
                       + jnp.dot(msg, Wu2, preferred_element_type=f32))
        H = jnp.where(maskb, new, H)

    G1 = jnp.sum((H * maskf).reshape(B, N, NODE_F), axis=1)
    G2 = jnp.sum((H0 * maskf).reshape(B, N, NODE_F), axis=1)
    out_ref[:] = (jnp.dot(G1, Wo1, preferred_element_type=f32)
                  + jnp.dot(G2, Wo2, preferred_element_type=f32))


def kernel(adjacency, nodes, edges, W_msg, W_upd, W_out):
    return pl.pallas_call(
        _mpnn_kernel,
        out_shape=jax.ShapeDtypeStruct((B, OUT_F), jnp.float32),
    )(adjacency.reshape(BN, N), nodes.reshape(BN, NODE_F),
      edges.reshape(E_ROWS, EDGE_F), W_msg, W_upd, W_out)
